# baseline (speedup 1.0000x reference)
import functools

import jax
import jax.numpy as jnp
from jax import lax
from jax.experimental import pallas as pl
from jax.experimental.pallas import tpu as pltpu

_B = 256
_M = 8192
_W = 32
_K = 4
_R = 5
_IN = 256
_BT = 8
_LG = _M // 4


def _scan_body(xi_ref, wt_ref, bi_ref, rw_ref, rp_ref, rv_ref, us_ref, mem_ref,
               idx1_ref, idx2_ref, ovr_ref, upd_ref):
    bt = _BT
    f32 = jnp.float32
    itf = jnp.dot(xi_ref[...], wt_ref[...], preferred_element_type=f32) + bi_ref[...]
    q = itf[:, 0:_W]
    wv = itf[:, _W:2 * _W]
    ig = 1.0 / (1.0 + jnp.exp(-itf[:, 2 * _W:2 * _W + _R]))
    wg = 1.0 / (1.0 + jnp.exp(-itf[:, 2 * _W + _R:2 * _W + _R + 1]))
    rp = rp_ref[...]
    rw = rw_ref[...]

    us = us_ref[...]
    lane_u = lax.broadcasted_iota(jnp.int32, (bt, _M), 1)
    ru_cols = []
    for j in range(_R):
        sel = lane_u == rp[:, j:j + 1]
        ru_cols.append(jnp.sum(jnp.where(sel, us, 0.0), axis=1, keepdims=True))
    ru = jnp.concatenate(ru_cols, axis=1)
    minu = jnp.min(ru, axis=1, keepdims=True)
    ind = (ru == minu).astype(f32)

    wwn = wg * (ig * rw + (1.0 - ig) * ind)
    lane160 = lax.broadcasted_iota(jnp.int32, (bt, _R * _W), 1)
    j_of_lane = lane160 // _W
    wv160 = jnp.concatenate([wv] * _R, axis=1)
    q160 = jnp.concatenate([q] * _R, axis=1)
    wwn160 = jnp.zeros((bt, _R * _W), f32)
    for j in range(_R):
        wwn160 = jnp.where(j_of_lane == j, wwn[:, j:j + 1], wwn160)
    rv_upd = rv_ref[...] + wwn160 * wv160
    upd_ref[...] = rv_upd
    dsq160 = (rv_upd - q160) ** 2
    dupd_cols = []
    for j in range(_R):
        dupd_cols.append(jnp.sum(jnp.where(j_of_lane == j, dsq160, 0.0),
                                 axis=1, keepdims=True))

    q128 = jnp.concatenate([q] * 4, axis=1)
    mem = mem_ref[...]
    d = mem - q128[:, None, :]
    t = d * d
    for sh in (16, 8, 4, 2, 1):
        t = t + pltpu.roll(t, 128 - sh, 2)
    lane_i = lax.broadcasted_iota(jnp.int32, (_LG, 128), 1)[None]
    sub_i = lax.broadcasted_iota(jnp.int32, (_LG, 128), 0)[None]
    valid = (lane_i % _W) == 0
    rid = sub_i * 4 + lane_i // _W
    big_i = jnp.int32(1 << 30)
    inf = jnp.float32(jnp.inf)
    sm = jnp.where(valid, t, inf)
    ridv = jnp.where(valid, rid, big_i)

    for j in range(_R):
        pj = rp[:, j:j + 1][:, :, None]
        dj = dupd_cols[j][:, :, None]
        sm = jnp.where(ridv == pj, dj, sm)

    pos_cols = []
    for _ in range(_K):
        m1 = jnp.min(sm, axis=1)
        vmin = jnp.min(m1, axis=1, keepdims=True)
        cand = jnp.where(sm == vmin[:, :, None], ridv, big_i)
        c1 = jnp.min(cand, axis=1)
        posk = jnp.min(c1, axis=1, keepdims=True)
        pos_cols.append(posk)
        sm = jnp.where(ridv == posk[:, :, None], inf, sm)
    pos = jnp.concatenate(pos_cols, axis=1)

    b0 = pl.program_id(0) * bt
    brow = b0 + lax.broadcasted_iota(jnp.int32, (bt, _K), 0)
    idx1_ref[...] = brow * _M + pos
    idx2 = jnp.zeros((bt, _K), jnp.int32)
    ovr = jnp.zeros((bt, _K), f32)
    for j in range(_R):
        match = pos == rp[:, j:j + 1]
        idx2 = jnp.where(match, brow * _R + j, idx2)
        ovr = jnp.where(match, 1.0, ovr)
    idx2_ref[...] = idx2
    lane128 = lax.broadcasted_iota(jnp.int32, (bt, 128), 1)
    k_of_lane = lane128 // _W
    ovr128 = jnp.zeros((bt, 128), f32)
    for k in range(_K):
        ovr128 = jnp.where(k_of_lane == k, ovr[:, k:k + 1], ovr128)
    ovr_ref[...] = ovr128


def _run_scan(xi, wt, bi, rw, rp, rv, usage, mem_packed, interpret=False):
    grid = _B // _BT
    bspec = lambda shape: pl.BlockSpec(shape, lambda i: (i,) + (0,) * (len(shape) - 1))
    return pl.pallas_call(
        _scan_body,
        grid=(grid,),
        in_specs=[
            bspec((_BT, _IN)),
            pl.BlockSpec((_IN, 2 * _W + _R + 1), lambda i: (0, 0)),
            pl.BlockSpec((1, 2 * _W + _R + 1), lambda i: (0, 0)),
            bspec((_BT, _R)),
            bspec((_BT, _R)),
            bspec((_BT, _R * _W)),
            bspec((_BT, _M)),
            bspec((_BT, _LG, 128)),
        ],
        out_specs=[
            bspec((_BT, _K)),
            bspec((_BT, _K)),
            bspec((_BT, 128)),
            bspec((_BT, _R * _W)),
        ],
        out_shape=[
            jax.ShapeDtypeStruct((_B, _K), jnp.int32),
            jax.ShapeDtypeStruct((_B, _K), jnp.int32),
            jax.ShapeDtypeStruct((_B, 128), jnp.float32),
            jax.ShapeDtypeStruct((_B, _R * _W), jnp.float32),
        ],
        interpret=interpret,
    )(xi, wt, bi, rw, rp, rv, usage, mem_packed)


def kernel(xi, memory, read_weights, write_weights, read_vectors, usage,
           W_interface, b_interface, last_used_mem, read_positions):
    del write_weights, last_used_mem
    wt = W_interface.T
    bi = b_interface.reshape(1, 2 * _W + _R + 1)
    rw = read_weights.reshape(_B, _R)
    rp = read_positions.reshape(_B, _R).astype(jnp.int32)
    rv = read_vectors.reshape(_B, _R * _W)
    mem_packed = memory.reshape(_B, _LG, 128)
    idx1, idx2, ovr, upd = _run_scan(xi, wt, bi, rw, rp, rv, usage, mem_packed)
    memflat = memory.reshape(_B * _M, _W)
    updflat = upd.reshape(_B * _R, _W)
    r1 = memflat[idx1.reshape(-1)]
    r2 = updflat[idx2.reshape(-1)]
    m = ovr.reshape(_B * _K, _W)
    out = jnp.where(m > 0.5, r2, r1)
    return out.reshape(_B, _K, _W)


# speedup vs baseline: 1.2416x; 1.2416x over previous
import jax
import jax.numpy as jnp
from jax import lax
from jax.experimental import pallas as pl
from jax.experimental.pallas import tpu as pltpu

_B = 256
_M = 8192
_W = 32
_K = 4
_R = 5
_IN = 256
_BT = 4
_NB = _B // _BT


def _scan_body(xi_ref, wt_ref, bi_ref, rw_ref, rp_ref, rv_ref, us_ref, mem_ref,
               pos_ref, idx2_ref, ovr_ref, upd_ref):
    bt = _BT
    f32 = jnp.float32
    itf = jnp.dot(xi_ref[0], wt_ref[...], preferred_element_type=f32) + bi_ref[...]
    q = itf[:, 0:_W]
    wv = itf[:, _W:2 * _W]
    ig = 1.0 / (1.0 + jnp.exp(-itf[:, 2 * _W:2 * _W + _R]))
    wg = 1.0 / (1.0 + jnp.exp(-itf[:, 2 * _W + _R:2 * _W + _R + 1]))
    rp = rp_ref[0]
    rw = rw_ref[0]

    us = us_ref[0]
    lane_u = lax.broadcasted_iota(jnp.int32, (bt, _M), 1)
    ru_cols = []
    for j in range(_R):
        sel = lane_u == rp[:, j:j + 1]
        ru_cols.append(jnp.sum(jnp.where(sel, us, 0.0), axis=1, keepdims=True))
    ru = jnp.concatenate(ru_cols, axis=1)
    minu = jnp.min(ru, axis=1, keepdims=True)
    ind = (ru == minu).astype(f32)

    wwn = wg * (ig * rw + (1.0 - ig) * ind)
    lane160 = lax.broadcasted_iota(jnp.int32, (bt, _R * _W), 1)
    j_of_lane = lane160 // _W
    wv160 = jnp.concatenate([wv] * _R, axis=1)
    q160 = jnp.concatenate([q] * _R, axis=1)
    wwn160 = jnp.zeros((bt, _R * _W), f32)
    for j in range(_R):
        wwn160 = jnp.where(j_of_lane == j, wwn[:, j:j + 1], wwn160)
    rv_upd = rv_ref[0] + wwn160 * wv160
    upd_ref[0] = rv_upd
    dsq160 = (rv_upd - q160) ** 2
    dupd_cols = []
    for j in range(_R):
        dupd_cols.append(jnp.sum(jnp.where(j_of_lane == j, dsq160, 0.0),
                                 axis=1, keepdims=True))

    mem = mem_ref[...]
    d = mem - q[:, None, :]
    s = jnp.sum(d * d, axis=2)

    for j in range(_R):
        s = jnp.where(lane_u == rp[:, j:j + 1], dupd_cols[j], s)

    big_i = jnp.int32(1 << 30)
    inf = jnp.float32(jnp.inf)
    pos_cols = []
    for _ in range(_K):
        vmin = jnp.min(s, axis=1, keepdims=True)
        cand = jnp.where(s == vmin, lane_u, big_i)
        posk = jnp.min(cand, axis=1, keepdims=True)
        pos_cols.append(posk)
        s = jnp.where(lane_u == posk, inf, s)
    pos = jnp.concatenate(pos_cols, axis=1)
    pos_ref[0] = pos

    b0 = pl.program_id(0) * bt
    brow = b0 + lax.broadcasted_iota(jnp.int32, (bt, _K), 0)
    idx2 = jnp.zeros((bt, _K), jnp.int32)
    ovr = jnp.zeros((bt, _K), f32)
    for j in range(_R):
        match = pos == rp[:, j:j + 1]
        idx2 = jnp.where(match, brow * _R + j, idx2)
        ovr = jnp.where(match, 1.0, ovr)
    idx2_ref[0] = idx2
    lane128 = lax.broadcasted_iota(jnp.int32, (bt, 128), 1)
    k_of_lane = lane128 // _W
    ovr128 = jnp.zeros((bt, 128), f32)
    for k in range(_K):
        ovr128 = jnp.where(k_of_lane == k, ovr[:, k:k + 1], ovr128)
    ovr_ref[0] = ovr128


def _run_scan(xi, wt, bi, rw, rp, rv, usage, memory, interpret=False):
    nfix = 2 * _W + _R + 1
    b3 = lambda shape: pl.BlockSpec((1,) + shape, lambda i: (i,) + (0,) * len(shape))
    return pl.pallas_call(
        _scan_body,
        grid=(_NB,),
        in_specs=[
            b3((_BT, _IN)),
            pl.BlockSpec((_IN, nfix), lambda i: (0, 0)),
            pl.BlockSpec((1, nfix), lambda i: (0, 0)),
            b3((_BT, _R)),
            b3((_BT, _R)),
            b3((_BT, _R * _W)),
            b3((_BT, _M)),
            pl.BlockSpec((_BT, _M, _W), lambda i: (i, 0, 0)),
        ],
        out_specs=[
            b3((_BT, _K)),
            b3((_BT, _K)),
            b3((_BT, 128)),
            b3((_BT, _R * _W)),
        ],
        out_shape=[
            jax.ShapeDtypeStruct((_NB, _BT, _K), jnp.int32),
            jax.ShapeDtypeStruct((_NB, _BT, _K), jnp.int32),
            jax.ShapeDtypeStruct((_NB, _BT, 128), jnp.float32),
            jax.ShapeDtypeStruct((_NB, _BT, _R * _W), jnp.float32),
        ],
        interpret=interpret,
    )(xi, wt, bi, rw, rp, rv, usage, memory)


def kernel(xi, memory, read_weights, write_weights, read_vectors, usage,
           W_interface, b_interface, last_used_mem, read_positions):
    del write_weights, last_used_mem
    nfix = 2 * _W + _R + 1
    wt = W_interface.T
    bi = b_interface.reshape(1, nfix)
    rw = read_weights.reshape(_NB, _BT, _R)
    rp = read_positions.reshape(_NB, _BT, _R).astype(jnp.int32)
    rv = read_vectors.reshape(_NB, _BT, _R * _W)
    xi3 = xi.reshape(_NB, _BT, _IN)
    us3 = usage.reshape(_NB, _BT, _M)
    pos, idx2, ovr, upd = _run_scan(xi3, wt, bi, rw, rp, rv, us3, memory)
    pos2 = pos.reshape(_B, _K)
    r1 = jnp.take_along_axis(memory, pos2[:, :, None], axis=1)
    updflat = upd.reshape(_B * _R, _W)
    r2 = updflat[idx2.reshape(-1)].reshape(_B, _K, _W)
    m = ovr.reshape(_B, _K, _W)
    return jnp.where(m > 0.5, r2, r1)
